# baseline (speedup 1.0000x reference)
import jax
import jax.numpy as jnp
from jax.experimental import pallas as pl
from jax.experimental.pallas import tpu as pltpu

EPS = 1e-6
BN_EPS = 1e-5
SLOPE = 0.2
B = 4
N = 1024
K = 20


def _mm3(h, W):
    return jnp.stack(
        [jnp.dot(h[i], W.T, preferred_element_type=jnp.float32) for i in range(3)],
        axis=0,
    )


def _vnl4(h, W, D):
    p = _mm3(h, W)
    d = _mm3(h, D)
    nrm = jnp.sqrt(p[0] ** 2 + p[1] ** 2 + p[2] ** 2) + EPS
    mean = jnp.mean(nrm, axis=0, keepdims=True)
    var = jnp.mean((nrm - mean) ** 2, axis=0, keepdims=True)
    nbn = (nrm - mean) / jnp.sqrt(var + BN_EPS)
    scale = nbn / nrm
    p = p * scale[None]
    dot = jnp.sum(p * d, axis=0)
    dsq = jnp.sum(d * d, axis=0)
    neg = (dot < 0).astype(p.dtype)
    coef = neg * dot / (dsq + EPS)
    return SLOPE * p + (1.0 - SLOPE) * (p - coef[None] * d)


def _tail_kernel(xc_ref, W5r, D5r, Wv1r, Dv1r, Wv2r, Dv2r, Wlinr, W6r, D6r,
                 out_ref):
    h = _vnl4(xc_ref[...], W5r[...], D5r[...])
    hm_parts = []
    for b in range(B):
        m = jnp.mean(h[:, b * N:(b + 1) * N, :], axis=1, keepdims=True)
        hm_parts.append(jnp.broadcast_to(m, (3, N, 341)))
    hm = jnp.concatenate(hm_parts, axis=1)
    h2 = jnp.concatenate([h, hm], axis=2)
    z = _vnl4(h2, Wv1r[...], Dv1r[...])
    z = _vnl4(z, Wv2r[...], Dv2r[...])
    z = _mm3(z, Wlinr[...])
    o = [sum(h2[c] * z[c, :, k][:, None] for c in range(3)) for k in range(3)]
    ho = jnp.stack(o, axis=0)
    ho = _vnl4(ho, W6r[...], D6r[...])
    outs = [jnp.mean(ho[:, b * N:(b + 1) * N, :], axis=1) for b in range(B)]
    out_ref[...] = jnp.stack(outs, axis=1)


def _tail(xc, W5, D5, Wv1, Dv1, Wv2, Dv2, Wlin, W6, D6):
    out = pl.pallas_call(
        _tail_kernel,
        out_shape=jax.ShapeDtypeStruct((3, B, 3), jnp.float32),
    )(xc, W5, D5, Wv1, Dv1, Wv2, Dv2, Wlin, W6, D6)
    return jnp.transpose(out, (1, 2, 0))


def _lin(x, W):
    return jnp.swapaxes(jnp.swapaxes(x, 1, -1) @ W.T, 1, -1)


def _vn_bn(norm, dim):
    axes = (0, 2, 3) if dim == 5 else (0, 2)
    mean = norm.mean(axis=axes, keepdims=True)
    var = ((norm - mean) ** 2).mean(axis=axes, keepdims=True)
    return (norm - mean) / jnp.sqrt(var + BN_EPS)


def _vn_lrelu(x, W, D, dim, slope=0.2):
    p = _lin(x, W)
    norm = jnp.linalg.norm(p, axis=2) + EPS
    norm_bn = _vn_bn(norm, dim)
    p = p / jnp.expand_dims(norm, 2) * jnp.expand_dims(norm_bn, 2)
    d = _lin(x, D)
    dot = jnp.sum(p * d, axis=2, keepdims=True)
    mask = (dot >= 0).astype(p.dtype)
    dsq = jnp.sum(d * d, axis=2, keepdims=True)
    return slope * p + (1 - slope) * (mask * p + (1 - mask) * (p - (dot / (dsq + EPS)) * d))


def _get_graph_feature(x, k):
    b, C, _, n = x.shape
    xf = x.reshape(b, C * 3, n)
    inner = -2.0 * jnp.einsum('bdn,bdm->bnm', xf, xf)
    xx = jnp.sum(xf ** 2, axis=1, keepdims=True)
    pdist = -xx - inner - jnp.swapaxes(xx, 1, 2)
    _, idx = jax.lax.top_k(pdist, k)
    xt = jnp.swapaxes(xf, 1, 2)
    feat = xt[jnp.arange(b)[:, None, None], idx]
    feat = feat.reshape(b, n, k, C, 3)
    xc = jnp.broadcast_to(xt.reshape(b, n, 1, C, 3), (b, n, k, C, 3))
    out = jnp.concatenate([feat - xc, xc], axis=3)
    return jnp.transpose(out, (0, 3, 4, 1, 2))


def _stage(x, W, D):
    h = _get_graph_feature(x, K)
    h = _vn_lrelu(h, W, D, 5)
    return h.mean(-1)


def kernel(x, W1, D1, W2, D2, W3, D3, W4, D4, W5, D5, Wv1, Dv1, Wv2, Dv2,
           Wlin, W6, D6):
    xx = jnp.swapaxes(x, 2, 1)[:, None]
    x1 = _stage(xx, W1, D1)
    x2 = _stage(x1, W2, D2)
    x3 = _stage(x2, W3, D3)
    x4 = _stage(x3, W4, D4)
    h = jnp.concatenate([x1, x2, x3, x4], axis=1)
    h = _vn_lrelu(h, W5, D5, 4)
    hm = jnp.broadcast_to(h.mean(-1, keepdims=True), h.shape)
    h = jnp.concatenate([h, hm], axis=1)
    z = _vn_lrelu(h, Wv1, Dv1, 4)
    z = _vn_lrelu(z, Wv2, Dv2, 4)
    z = _lin(z, Wlin)
    z = jnp.swapaxes(z, 1, 2)
    h = jnp.einsum('bijm,bjkm->bikm', h, z)
    h = _vn_lrelu(h, W6, D6, 4)
    return h.mean(-1)


# speedup vs baseline: 2.9002x; 2.9002x over previous
import functools

import jax
import jax.numpy as jnp
from jax.experimental import pallas as pl
from jax.experimental.pallas import tpu as pltpu

EPS = 1e-6
BN_EPS = 1e-5
SLOPE = 0.2
B = 4
N = 1024
K = 20
BLKN = 256
NB = N // BLKN
E = BLKN * K
NEGBIG = -1e30


HIGHEST = jax.lax.Precision.HIGHEST


def _dotT(a, b):
    return jax.lax.dot_general(a, b, (((1,), (1,)), ((), ())),
                               precision=HIGHEST,
                               preferred_element_type=jnp.float32)


def _dot(a, b):
    return jnp.dot(a, b, precision=HIGHEST,
                   preferred_element_type=jnp.float32)


def _stageA_kernel(O, C, xs_ref, Wg3_ref, Wc3_ref, feat_ref, stats_ref):
    nb = pl.program_id(1)
    xb = xs_ref[0]
    xr = xs_ref[0, pl.ds(nb * BLKN, BLKN), :]

    g = _dotT(xr, xb)
    inner = -2.0 * g
    sq_r = jnp.sum(xr * xr, axis=1, keepdims=True)
    sq_c = _dotT(jnp.ones((1, xb.shape[1]), jnp.float32), xb * xb)
    pd = (-sq_r) - inner - sq_c

    iota = jax.lax.broadcasted_iota(jnp.int32, (BLKN, N), 1)
    for t in range(K):
        m = jnp.max(pd, axis=1, keepdims=True)
        cand = jnp.where(pd >= m, iota, N)
        j = jnp.min(cand, axis=1, keepdims=True)
        mask = iota == j
        mf = mask.astype(jnp.float32)
        feat_ref[0, t * BLKN:(t + 1) * BLKN, :] = _dot(mf, xb)
        pd = jnp.where(mask, NEGBIG, pd)

    feat = feat_ref[0]
    cw = _dot(xr, Wc3_ref[...])
    p = _dot(feat, Wg3_ref[...]) + jnp.concatenate([cw] * K, axis=0)
    nsq = sum(p[:, c * O:(c + 1) * O] ** 2 for c in range(3))
    nrm = jnp.sqrt(nsq) + EPS
    s1 = jnp.sum(nrm, axis=0, keepdims=True)
    s2 = jnp.sum(nrm * nrm, axis=0, keepdims=True)
    stats_ref[...] = jnp.concatenate([s1, s2], axis=1)[None]


def _stageC_kernel(O, C, xs_ref, feat_ref, stats_ref, Wg3_ref, Wc3_ref,
                   Dg3_ref, Dc3_ref, out_ref):
    st = jnp.sum(stats_ref[...], axis=0)
    cnt = float(B * N * K)
    mean = st[:, :O] / cnt
    ex2 = st[:, O:] / cnt
    var = ex2 - mean * mean
    inv = jax.lax.rsqrt(var + BN_EPS)

    xr = xs_ref[0]
    feat = feat_ref[0]
    cw = _dot(xr, Wc3_ref[...])
    cd = _dot(xr, Dc3_ref[...])
    p = _dot(feat, Wg3_ref[...]) + jnp.concatenate([cw] * K, axis=0)
    d = _dot(feat, Dg3_ref[...]) + jnp.concatenate([cd] * K, axis=0)

    nsq = sum(p[:, c * O:(c + 1) * O] ** 2 for c in range(3))
    nrm = jnp.sqrt(nsq) + EPS
    nbn = (nrm - mean) * inv
    scale = nbn / nrm
    p = p * jnp.concatenate([scale] * 3, axis=1)

    q = p * d
    dot = sum(q[:, c * O:(c + 1) * O] for c in range(3))
    dd = d * d
    dsq = sum(dd[:, c * O:(c + 1) * O] for c in range(3))
    neg = (dot < 0).astype(jnp.float32)
    coef = neg * dot / (dsq + EPS)
    o = SLOPE * p + (1.0 - SLOPE) * (p - jnp.concatenate([coef] * 3, axis=1) * d)

    acc = o[0:BLKN, :]
    for t in range(1, K):
        acc = acc + o[t * BLKN:(t + 1) * BLKN, :]
    out_ref[0] = acc * (1.0 / K)


def _stage(xs, W, D):
    C2 = W.shape[1]
    C = C2 // 2
    O = W.shape[0]
    f32 = jnp.float32
    WaW = W[:, :C].T
    WcW = (W[:, C:] - W[:, :C]).T
    WaD = D[:, :C].T
    WcD = (D[:, C:] - D[:, :C]).T
    bd = jax.scipy.linalg.block_diag
    Wg3 = bd(WaW, WaW, WaW)
    Wc3 = bd(WcW, WcW, WcW)
    Dg3 = bd(WaD, WaD, WaD)
    Dc3 = bd(WcD, WcD, WcD)

    feat, stats = pl.pallas_call(
        functools.partial(_stageA_kernel, O, C),
        grid=(B, NB),
        in_specs=[
            pl.BlockSpec((1, N, 3 * C), lambda b, nb: (b, 0, 0)),
            pl.BlockSpec((3 * C, 3 * O), lambda b, nb: (0, 0)),
            pl.BlockSpec((3 * C, 3 * O), lambda b, nb: (0, 0)),
        ],
        out_specs=[
            pl.BlockSpec((1, E, 3 * C), lambda b, nb: (b, nb, 0)),
            pl.BlockSpec((1, 1, 2 * O), lambda b, nb: (b * NB + nb, 0, 0)),
        ],
        out_shape=[
            jax.ShapeDtypeStruct((B, N * K, 3 * C), f32),
            jax.ShapeDtypeStruct((B * NB, 1, 2 * O), f32),
        ],
    )(xs, Wg3, Wc3)

    out = pl.pallas_call(
        functools.partial(_stageC_kernel, O, C),
        grid=(B, NB),
        in_specs=[
            pl.BlockSpec((1, BLKN, 3 * C), lambda b, nb: (b, nb, 0)),
            pl.BlockSpec((1, E, 3 * C), lambda b, nb: (b, nb, 0)),
            pl.BlockSpec((B * NB, 1, 2 * O), lambda b, nb: (0, 0, 0)),
            pl.BlockSpec((3 * C, 3 * O), lambda b, nb: (0, 0)),
            pl.BlockSpec((3 * C, 3 * O), lambda b, nb: (0, 0)),
            pl.BlockSpec((3 * C, 3 * O), lambda b, nb: (0, 0)),
            pl.BlockSpec((3 * C, 3 * O), lambda b, nb: (0, 0)),
        ],
        out_specs=pl.BlockSpec((1, BLKN, 3 * O), lambda b, nb: (b, nb, 0)),
        out_shape=jax.ShapeDtypeStruct((B, N, 3 * O), f32),
    )(xs, feat, stats, Wg3, Wc3, Dg3, Dc3)
    return out


R = B * N


def _mmp(parts, Wts):
    acc = _dot(parts[0], Wts[0])
    for hpart, wt in zip(parts[1:], Wts[1:]):
        acc = acc + _dot(hpart, wt)
    return acc


def _vnlA_kernel(nparts, *refs):
    h_refs = refs[:nparts]
    w_refs = refs[nparts:2 * nparts]
    norm_ref = refs[-1]
    p = [None] * 3
    for cm in range(3):
        p[cm] = _mmp([r[cm] for r in h_refs], [w[...] for w in w_refs])
    norm_ref[...] = jnp.sqrt(p[0] ** 2 + p[1] ** 2 + p[2] ** 2) + EPS


def _vnlC_kernel(nparts, mean_out, *refs):
    h_refs = refs[:nparts]
    w_refs = refs[nparts:2 * nparts]
    d_refs = refs[2 * nparts:3 * nparts]
    nf_ref = refs[3 * nparts]
    out_ref = refs[-1]
    nf = nf_ref[...]
    mean = jnp.mean(nf, axis=0, keepdims=True)
    var = jnp.mean((nf - mean) ** 2, axis=0, keepdims=True)
    inv = jax.lax.rsqrt(var + BN_EPS)

    hs = [r[...] for r in h_refs]
    p = [_mmp([h[cm] for h in hs], [w[...] for w in w_refs]) for cm in range(3)]
    d = [_mmp([h[cm] for h in hs], [w[...] for w in d_refs]) for cm in range(3)]
    nrm = jnp.sqrt(p[0] ** 2 + p[1] ** 2 + p[2] ** 2) + EPS
    scale = (nrm - mean) * inv / nrm
    p = [pc * scale for pc in p]
    dot = p[0] * d[0] + p[1] * d[1] + p[2] * d[2]
    dsq = d[0] * d[0] + d[1] * d[1] + d[2] * d[2]
    neg = (dot < 0).astype(jnp.float32)
    coef = neg * dot / (dsq + EPS)
    o = [SLOPE * pc + (1.0 - SLOPE) * (pc - coef * dc) for pc, dc in zip(p, d)]
    out = jnp.stack(o, axis=0)
    if mean_out:
        out_ref[0] = jnp.mean(out, axis=1)
    else:
        out_ref[...] = out


def _vnl_tail(parts, Wts, Dts, mean_out=False, tblk=256):
    np_ = len(parts)
    O = Wts[0].shape[1]
    f32 = jnp.float32
    nblk = R // tblk
    hspecs = [pl.BlockSpec((3, tblk, p.shape[2]), lambda i: (0, i, 0))
              for p in parts]
    wspecs = [pl.BlockSpec(w.shape, lambda i: (0, 0)) for w in Wts]
    dspecs = [pl.BlockSpec(dw.shape, lambda i: (0, 0)) for dw in Dts]

    norm = pl.pallas_call(
        functools.partial(_vnlA_kernel, np_),
        grid=(nblk,),
        in_specs=hspecs + wspecs,
        out_specs=pl.BlockSpec((tblk, O), lambda i: (i, 0)),
        out_shape=jax.ShapeDtypeStruct((R, O), f32),
    )(*parts, *Wts)

    if mean_out:
        ospec = pl.BlockSpec((1, 3, O), lambda i: (i, 0, 0))
        oshape = jax.ShapeDtypeStruct((nblk, 3, O), f32)
    else:
        ospec = pl.BlockSpec((3, tblk, O), lambda i: (0, i, 0))
        oshape = jax.ShapeDtypeStruct((3, R, O), f32)
    out = pl.pallas_call(
        functools.partial(_vnlC_kernel, np_, mean_out),
        grid=(nblk,),
        in_specs=hspecs + wspecs + dspecs
                 + [pl.BlockSpec((R, O), lambda i: (0, 0))],
        out_specs=ospec,
        out_shape=oshape,
    )(*parts, *Wts, *Dts, norm)
    return out


def _einsum_kernel(h_ref, hm_ref, z2_ref, wlin_ref, hoA_ref, hoB_ref):
    z3 = [None] * 3
    for cm in range(3):
        z3[cm] = _dot(z2_ref[cm], wlin_ref[...])
    h = h_ref[...]
    hm = hm_ref[...]
    for kk in range(3):
        a = sum(h[cm] * z3[cm][:, kk][:, None] for cm in range(3))
        bpart = sum(hm[cm] * z3[cm][:, kk][:, None] for cm in range(3))
        hoA_ref[kk] = a
        hoB_ref[kk] = bpart


def _einsum_call(h, hm_full, z2, Wlint, tblk=256):
    f32 = jnp.float32
    Ch = h.shape[2]
    return pl.pallas_call(
        _einsum_kernel,
        grid=(R // tblk,),
        in_specs=[
            pl.BlockSpec((3, tblk, Ch), lambda i: (0, i, 0)),
            pl.BlockSpec((3, tblk, Ch), lambda i: (0, i, 0)),
            pl.BlockSpec((3, tblk, z2.shape[2]), lambda i: (0, i, 0)),
            pl.BlockSpec(Wlint.shape, lambda i: (0, 0)),
        ],
        out_specs=[
            pl.BlockSpec((3, tblk, Ch), lambda i: (0, i, 0)),
            pl.BlockSpec((3, tblk, Ch), lambda i: (0, i, 0)),
        ],
        out_shape=[
            jax.ShapeDtypeStruct((3, R, Ch), f32),
            jax.ShapeDtypeStruct((3, R, Ch), f32),
        ],
    )(h, hm_full, z2, Wlint)


def kernel(x, W1, D1, W2, D2, W3, D3, W4, D4, W5, D5, Wv1, Dv1, Wv2, Dv2,
           Wlin, W6, D6):
    x1 = _stage(x, W1, D1)
    x2 = _stage(x1, W2, D2)
    x3 = _stage(x2, W3, D3)
    x4 = _stage(x3, W4, D4)

    def to_cm(xi, O):
        return jnp.transpose(xi.reshape(B, N, 3, O), (2, 0, 1, 3)).reshape(3, R, O)

    xcat = jnp.concatenate(
        [to_cm(x1, 21), to_cm(x2, 21), to_cm(x3, 42), to_cm(x4, 85)], axis=2)

    h = _vnl_tail([xcat], [W5.T], [D5.T])
    hm = jnp.mean(h.reshape(3, B, N, 341), axis=2, keepdims=True)
    hm_full = jnp.broadcast_to(hm, (3, B, N, 341)).reshape(3, R, 341)
    z1 = _vnl_tail([h, hm_full],
                   [Wv1[:, :341].T, Wv1[:, 341:].T],
                   [Dv1[:, :341].T, Dv1[:, 341:].T])
    z2 = _vnl_tail([z1], [Wv2.T], [Dv2.T])
    hoA, hoB = _einsum_call(h, hm_full, z2, Wlin.T)
    om = _vnl_tail([hoA, hoB],
                   [W6[:, :341].T, W6[:, 341:].T],
                   [D6[:, :341].T, D6[:, 341:].T],
                   mean_out=True)
    om = jnp.mean(om.reshape(B, 4, 3, 3), axis=1)
    return jnp.transpose(om, (0, 2, 1))


# speedup vs baseline: 4.0167x; 1.3850x over previous
import functools

import jax
import jax.numpy as jnp
from jax.experimental import pallas as pl
from jax.experimental.pallas import tpu as pltpu

EPS = 1e-6
BN_EPS = 1e-5
SLOPE = 0.2
B = 4
N = 1024
K = 20
BLKN = 256
NB = N // BLKN
E = BLKN * K
NEGBIG = -1e30


BF16 = jnp.bfloat16


def _split(x):
    hi = x.astype(BF16)
    lo = (x - hi.astype(jnp.float32)).astype(BF16)
    return hi, lo


def _dotbf(a, b):
    return jnp.dot(a, b, preferred_element_type=jnp.float32)


def _dotbfT(a, b):
    return jax.lax.dot_general(a, b, (((1,), (1,)), ((), ())),
                               preferred_element_type=jnp.float32)


def _dot1(a, b):
    return _dotbf(a.astype(BF16), b.astype(BF16))


def _dot3(a, b):
    ahi, alo = _split(a)
    bhi, blo = _split(b)
    return _dotbf(ahi, bhi) + (_dotbf(ahi, blo) + _dotbf(alo, bhi))


def _dotT3(a, b):
    ahi, alo = _split(a)
    bhi, blo = _split(b)
    return _dotbfT(ahi, bhi) + (_dotbfT(ahi, blo) + _dotbfT(alo, bhi))


def _split3(x):
    hi = x.astype(BF16)
    r = x - hi.astype(jnp.float32)
    mid = r.astype(BF16)
    lo = (r - mid.astype(jnp.float32)).astype(BF16)
    return hi, mid, lo


HIGHEST = jax.lax.Precision.HIGHEST


def _dotH(a, b):
    return jnp.dot(a, b, precision=HIGHEST, preferred_element_type=jnp.float32)


def _dotHT(a, b):
    return jax.lax.dot_general(a, b, (((1,), (1,)), ((), ())),
                               precision=HIGHEST,
                               preferred_element_type=jnp.float32)


def _stageA_kernel(O, C, xs_ref, Wg3_ref, Wc3_ref, feat_ref, stats_ref):
    nb = pl.program_id(1)
    xb = xs_ref[0]
    xr = xs_ref[0, pl.ds(nb * BLKN, BLKN), :]

    g = _dotHT(xr, xb)
    inner = -2.0 * g
    sq_r = jnp.sum(xr * xr, axis=1, keepdims=True)
    sq_c = _dotHT(jnp.ones((1, xb.shape[1]), jnp.float32), xb * xb)
    pd = (-sq_r) - inner - sq_c

    xhi, xmid, xlo = _split3(xb)
    iota = jax.lax.broadcasted_iota(jnp.int32, (BLKN, N), 1)
    for t in range(K):
        m = jnp.max(pd, axis=1, keepdims=True)
        cand = jnp.where(pd >= m, iota, N)
        j = jnp.min(cand, axis=1, keepdims=True)
        mask = iota == j
        mf = mask.astype(BF16)
        feat_ref[0, t * BLKN:(t + 1) * BLKN, :] = (
            (_dotbf(mf, xhi) + _dotbf(mf, xmid)) + _dotbf(mf, xlo))
        pd = jnp.where(mask, NEGBIG, pd)

    feat = feat_ref[0]
    cw = _dot3(xr, Wc3_ref[...])
    p = _dot3(feat, Wg3_ref[...]) + jnp.concatenate([cw] * K, axis=0)
    nsq = sum(p[:, c * O:(c + 1) * O] ** 2 for c in range(3))
    nrm = jnp.sqrt(nsq) + EPS
    s1 = jnp.sum(nrm, axis=0, keepdims=True)
    s2 = jnp.sum(nrm * nrm, axis=0, keepdims=True)
    stats_ref[...] = jnp.concatenate([s1, s2], axis=1)[None]


def _stageC_kernel(O, C, xs_ref, feat_ref, stats_ref, Wg3_ref, Wc3_ref,
                   Dg3_ref, Dc3_ref, out_ref):
    st = jnp.sum(stats_ref[...], axis=0)
    cnt = float(B * N * K)
    mean = st[:, :O] / cnt
    ex2 = st[:, O:] / cnt
    var = ex2 - mean * mean
    inv = jax.lax.rsqrt(var + BN_EPS)

    xr = xs_ref[0]
    feat = feat_ref[0]
    cw = _dotH(xr, Wc3_ref[...])
    cd = _dotH(xr, Dc3_ref[...])
    p = _dotH(feat, Wg3_ref[...]) + jnp.concatenate([cw] * K, axis=0)
    d = _dotH(feat, Dg3_ref[...]) + jnp.concatenate([cd] * K, axis=0)

    nsq = sum(p[:, c * O:(c + 1) * O] ** 2 for c in range(3))
    nrm = jnp.sqrt(nsq) + EPS
    nbn = (nrm - mean) * inv
    scale = nbn / nrm
    p = p * jnp.concatenate([scale] * 3, axis=1)

    q = p * d
    dot = sum(q[:, c * O:(c + 1) * O] for c in range(3))
    dd = d * d
    dsq = sum(dd[:, c * O:(c + 1) * O] for c in range(3))
    neg = (dot < 0).astype(jnp.float32)
    coef = neg * dot / (dsq + EPS)
    o = SLOPE * p + (1.0 - SLOPE) * (p - jnp.concatenate([coef] * 3, axis=1) * d)

    acc = o[0:BLKN, :]
    for t in range(1, K):
        acc = acc + o[t * BLKN:(t + 1) * BLKN, :]
    out_ref[0] = acc * (1.0 / K)


def _stage(xs, W, D):
    C2 = W.shape[1]
    C = C2 // 2
    O = W.shape[0]
    f32 = jnp.float32
    WaW = W[:, :C].T
    WcW = (W[:, C:] - W[:, :C]).T
    WaD = D[:, :C].T
    WcD = (D[:, C:] - D[:, :C]).T
    bd = jax.scipy.linalg.block_diag
    Wg3 = bd(WaW, WaW, WaW)
    Wc3 = bd(WcW, WcW, WcW)
    Dg3 = bd(WaD, WaD, WaD)
    Dc3 = bd(WcD, WcD, WcD)

    feat, stats = pl.pallas_call(
        functools.partial(_stageA_kernel, O, C),
        grid=(B, NB),
        in_specs=[
            pl.BlockSpec((1, N, 3 * C), lambda b, nb: (b, 0, 0)),
            pl.BlockSpec((3 * C, 3 * O), lambda b, nb: (0, 0)),
            pl.BlockSpec((3 * C, 3 * O), lambda b, nb: (0, 0)),
        ],
        out_specs=[
            pl.BlockSpec((1, E, 3 * C), lambda b, nb: (b, nb, 0)),
            pl.BlockSpec((1, 1, 2 * O), lambda b, nb: (b * NB + nb, 0, 0)),
        ],
        out_shape=[
            jax.ShapeDtypeStruct((B, N * K, 3 * C), f32),
            jax.ShapeDtypeStruct((B * NB, 1, 2 * O), f32),
        ],
    )(xs, Wg3, Wc3)

    out = pl.pallas_call(
        functools.partial(_stageC_kernel, O, C),
        grid=(B, NB),
        in_specs=[
            pl.BlockSpec((1, BLKN, 3 * C), lambda b, nb: (b, nb, 0)),
            pl.BlockSpec((1, E, 3 * C), lambda b, nb: (b, nb, 0)),
            pl.BlockSpec((B * NB, 1, 2 * O), lambda b, nb: (0, 0, 0)),
            pl.BlockSpec((3 * C, 3 * O), lambda b, nb: (0, 0)),
            pl.BlockSpec((3 * C, 3 * O), lambda b, nb: (0, 0)),
            pl.BlockSpec((3 * C, 3 * O), lambda b, nb: (0, 0)),
            pl.BlockSpec((3 * C, 3 * O), lambda b, nb: (0, 0)),
        ],
        out_specs=pl.BlockSpec((1, BLKN, 3 * O), lambda b, nb: (b, nb, 0)),
        out_shape=jax.ShapeDtypeStruct((B, N, 3 * O), f32),
    )(xs, feat, stats, Wg3, Wc3, Dg3, Dc3)
    return out


R = B * N


def _mmp(parts, Wts, dot_fn=_dot3):
    acc = dot_fn(parts[0], Wts[0])
    for hpart, wt in zip(parts[1:], Wts[1:]):
        acc = acc + dot_fn(hpart, wt)
    return acc


def _vnlA_kernel(nparts, *refs):
    h_refs = refs[:nparts]
    w_refs = refs[nparts:2 * nparts]
    norm_ref = refs[-1]
    p = [None] * 3
    for cm in range(3):
        p[cm] = _mmp([r[cm] for r in h_refs], [w[...] for w in w_refs])
    norm_ref[...] = jnp.sqrt(p[0] ** 2 + p[1] ** 2 + p[2] ** 2) + EPS


def _vnlC_kernel(nparts, mean_out, *refs):
    h_refs = refs[:nparts]
    w_refs = refs[nparts:2 * nparts]
    d_refs = refs[2 * nparts:3 * nparts]
    nf_ref = refs[3 * nparts]
    out_ref = refs[-1]
    nf = nf_ref[...]
    mean = jnp.mean(nf, axis=0, keepdims=True)
    var = jnp.mean((nf - mean) ** 2, axis=0, keepdims=True)
    inv = jax.lax.rsqrt(var + BN_EPS)

    hs = [r[...] for r in h_refs]
    p = [_mmp([h[cm] for h in hs], [w[...] for w in w_refs]) for cm in range(3)]
    d = [_mmp([h[cm] for h in hs], [w[...] for w in d_refs]) for cm in range(3)]
    nrm = jnp.sqrt(p[0] ** 2 + p[1] ** 2 + p[2] ** 2) + EPS
    scale = (nrm - mean) * inv / nrm
    p = [pc * scale for pc in p]
    dot = p[0] * d[0] + p[1] * d[1] + p[2] * d[2]
    dsq = d[0] * d[0] + d[1] * d[1] + d[2] * d[2]
    neg = (dot < 0).astype(jnp.float32)
    coef = neg * dot / (dsq + EPS)
    o = [SLOPE * pc + (1.0 - SLOPE) * (pc - coef * dc) for pc, dc in zip(p, d)]
    out = jnp.stack(o, axis=0)
    if mean_out:
        out_ref[0] = jnp.mean(out, axis=1)
    else:
        out_ref[...] = out


def _vnl_tail(parts, Wts, Dts, mean_out=False, tblk=256):
    np_ = len(parts)
    O = Wts[0].shape[1]
    f32 = jnp.float32
    nblk = R // tblk
    hspecs = [pl.BlockSpec((3, tblk, p.shape[2]), lambda i: (0, i, 0))
              for p in parts]
    wspecs = [pl.BlockSpec(w.shape, lambda i: (0, 0)) for w in Wts]
    dspecs = [pl.BlockSpec(dw.shape, lambda i: (0, 0)) for dw in Dts]

    norm = pl.pallas_call(
        functools.partial(_vnlA_kernel, np_),
        grid=(nblk,),
        in_specs=hspecs + wspecs,
        out_specs=pl.BlockSpec((tblk, O), lambda i: (i, 0)),
        out_shape=jax.ShapeDtypeStruct((R, O), f32),
    )(*parts, *Wts)

    if mean_out:
        ospec = pl.BlockSpec((1, 3, O), lambda i: (i, 0, 0))
        oshape = jax.ShapeDtypeStruct((nblk, 3, O), f32)
    else:
        ospec = pl.BlockSpec((3, tblk, O), lambda i: (0, i, 0))
        oshape = jax.ShapeDtypeStruct((3, R, O), f32)
    out = pl.pallas_call(
        functools.partial(_vnlC_kernel, np_, mean_out),
        grid=(nblk,),
        in_specs=hspecs + wspecs + dspecs
                 + [pl.BlockSpec((R, O), lambda i: (0, 0))],
        out_specs=ospec,
        out_shape=oshape,
    )(*parts, *Wts, *Dts, norm)
    return out


def _einsum_kernel(h_ref, hm_ref, z2_ref, wlin_ref, hoA_ref, hoB_ref):
    z3 = [None] * 3
    for cm in range(3):
        z3[cm] = _dot3(z2_ref[cm], wlin_ref[...])
    h = h_ref[...]
    hm = hm_ref[...]
    for kk in range(3):
        a = sum(h[cm] * z3[cm][:, kk][:, None] for cm in range(3))
        bpart = sum(hm[cm] * z3[cm][:, kk][:, None] for cm in range(3))
        hoA_ref[kk] = a
        hoB_ref[kk] = bpart


def _einsum_call(h, hm_full, z2, Wlint, tblk=256):
    f32 = jnp.float32
    Ch = h.shape[2]
    return pl.pallas_call(
        _einsum_kernel,
        grid=(R // tblk,),
        in_specs=[
            pl.BlockSpec((3, tblk, Ch), lambda i: (0, i, 0)),
            pl.BlockSpec((3, tblk, Ch), lambda i: (0, i, 0)),
            pl.BlockSpec((3, tblk, z2.shape[2]), lambda i: (0, i, 0)),
            pl.BlockSpec(Wlint.shape, lambda i: (0, 0)),
        ],
        out_specs=[
            pl.BlockSpec((3, tblk, Ch), lambda i: (0, i, 0)),
            pl.BlockSpec((3, tblk, Ch), lambda i: (0, i, 0)),
        ],
        out_shape=[
            jax.ShapeDtypeStruct((3, R, Ch), f32),
            jax.ShapeDtypeStruct((3, R, Ch), f32),
        ],
    )(h, hm_full, z2, Wlint)


def kernel(x, W1, D1, W2, D2, W3, D3, W4, D4, W5, D5, Wv1, Dv1, Wv2, Dv2,
           Wlin, W6, D6):
    x1 = _stage(x, W1, D1)
    x2 = _stage(x1, W2, D2)
    x3 = _stage(x2, W3, D3)
    x4 = _stage(x3, W4, D4)

    def to_cm(xi, O):
        return jnp.transpose(xi.reshape(B, N, 3, O), (2, 0, 1, 3)).reshape(3, R, O)

    xcat = jnp.concatenate(
        [to_cm(x1, 21), to_cm(x2, 21), to_cm(x3, 42), to_cm(x4, 85)], axis=2)

    h = _vnl_tail([xcat], [W5.T], [D5.T])
    hm = jnp.mean(h.reshape(3, B, N, 341), axis=2, keepdims=True)
    hm_full = jnp.broadcast_to(hm, (3, B, N, 341)).reshape(3, R, 341)
    z1 = _vnl_tail([h, hm_full],
                   [Wv1[:, :341].T, Wv1[:, 341:].T],
                   [Dv1[:, :341].T, Dv1[:, 341:].T])
    z2 = _vnl_tail([z1], [Wv2.T], [Dv2.T])
    hoA, hoB = _einsum_call(h, hm_full, z2, Wlin.T)
    om = _vnl_tail([hoA, hoB],
                   [W6[:, :341].T, W6[:, 341:].T],
                   [D6[:, :341].T, D6[:, 341:].T],
                   mean_out=True)
    om = jnp.mean(om.reshape(B, 4, 3, 3), axis=1)
    return jnp.transpose(om, (0, 2, 1))
